# baseline (speedup 1.0000x reference)
import functools

import jax
import jax.numpy as jnp
from jax import lax
from jax.experimental import pallas as pl
from jax.experimental.pallas import tpu as pltpu
from jax.experimental.pallas import tpu_sc as plsc

N = 10000
E = 320000
F_IN = 128
H = 64

NC = 2
NS = 16
NW = NC * NS
CHUNK = 128
CPT = 80
EPT = CPT * CHUNK
E_PAD = NW * EPT
N_ACC = 10240
RPT = N_ACC // NS
DUMMY = N

_mesh = plsc.VectorSubcoreMesh(
    core_axis_name="c", subcore_axis_name="s", num_cores=NC, num_subcores=NS)
_sc_params = pltpu.CompilerParams(use_tc_tiling_on_sc=False)


@functools.partial(
    pl.kernel,
    out_type=jax.ShapeDtypeStruct((NC, N_ACC), jnp.float32),
    mesh=_mesh,
    compiler_params=_sc_params,
    scratch_types=[
        pltpu.VMEM((CPT, CHUNK), jnp.int32),
        pltpu.VMEM((CHUNK,), jnp.float32),
        pltpu.VMEM_SHARED((N_ACC,), jnp.float32),
    ],
)
def _sc_degree(dst_hbm, zeros_hbm, out_hbm, dst_v, ones_v, deg_tab):
    c = lax.axis_index("c")
    s = lax.axis_index("s")
    wid = c * NS + s
    pltpu.sync_copy(zeros_hbm.at[pl.ds(s * RPT, RPT)],
                    deg_tab.at[pl.ds(s * RPT, RPT)])
    pltpu.sync_copy(dst_hbm.at[pl.ds(wid * CPT, CPT)], dst_v)
    for i in range(CHUNK // 16):
        ones_v[pl.ds(i * 16, 16)] = jnp.ones((16,), jnp.float32)
    plsc.subcore_barrier()

    def body(j, carry):
        pltpu.sync_copy(ones_v, deg_tab.at[dst_v.at[j]], add=True)
        return carry

    lax.fori_loop(0, CPT, body, 0)
    plsc.subcore_barrier()
    pltpu.sync_copy(deg_tab.at[pl.ds(s * RPT, RPT)],
                    out_hbm.at[c, pl.ds(s * RPT, RPT)])


NBUF = 8
TOT_CHUNKS = E_PAD // CHUNK
K0 = 40
K1 = (TOT_CHUNKS - NS * K0) // NS
KMAX = max(K0, K1)


@functools.partial(
    pl.kernel,
    out_type=jax.ShapeDtypeStruct((NC, N_ACC, H), jnp.float32),
    mesh=_mesh,
    compiler_params=_sc_params,
    scratch_types=[
        pltpu.VMEM((NBUF, CHUNK), jnp.int32),
        pltpu.VMEM((NBUF, CHUNK), jnp.int32),
        pltpu.VMEM((NBUF, CHUNK, H), jnp.float32),
        pltpu.VMEM_SHARED((N_ACC, H), jnp.float32),
    ] + [pltpu.SemaphoreType.DMA] * NBUF,
)
def _sc_edges(src_hbm, dst_hbm, g_hbm, zeros_hbm, out_hbm,
              src_v, dst_v, bufs, acc_tab, *sems):
    c = lax.axis_index("c")
    s = lax.axis_index("s")
    base = jnp.where(c == 0, s * K0, NS * K0 + s * K1)
    cnt = jnp.where(c == 0, K0, K1)
    pltpu.sync_copy(zeros_hbm.at[pl.ds(s * RPT, RPT)],
                    acc_tab.at[pl.ds(s * RPT, RPT)])
    plsc.subcore_barrier()

    def body(t, carry):
        gb = base + t * NBUF
        pltpu.sync_copy(src_hbm.at[pl.ds(gb, NBUF)], src_v)
        pltpu.sync_copy(dst_hbm.at[pl.ds(gb, NBUF)], dst_v)
        descs = [pltpu.async_copy(g_hbm.at[src_v.at[b]],
                                  bufs.at[b], sems[b])
                 for b in range(NBUF)]
        for b in range(NBUF):
            descs[b].wait()
            pltpu.sync_copy(bufs.at[b],
                            acc_tab.at[dst_v.at[b]], add=True)
        return carry

    lax.fori_loop(0, cnt // NBUF, body, 0)
    plsc.subcore_barrier()
    pltpu.sync_copy(acc_tab.at[pl.ds(s * RPT, RPT)],
                    out_hbm.at[c, pl.ds(s * RPT, RPT)])


def _prescale_body(x_ref, wc_ref, deg_ref, g_ref, dinv_ref):
    deg = deg_ref[:, 0] + deg_ref[:, 1] + 1.0
    dinv = lax.rsqrt(deg)
    h = jnp.dot(x_ref[...], wc_ref[...], preferred_element_type=jnp.float32)
    g_ref[...] = h * dinv[:, None]
    dinv_ref[...] = dinv[:, None]


def _tc_prescale(x, wc, deg2):
    blk = 1000
    grid = (N // blk,)
    return pl.pallas_call(
        _prescale_body,
        grid=grid,
        in_specs=[
            pl.BlockSpec((blk, F_IN), lambda i: (i, 0)),
            pl.BlockSpec((F_IN, H), lambda i: (0, 0)),
            pl.BlockSpec((blk, 2), lambda i: (i, 0)),
        ],
        out_specs=[
            pl.BlockSpec((blk, H), lambda i: (i, 0)),
            pl.BlockSpec((blk, 1), lambda i: (i, 0)),
        ],
        out_shape=[
            jax.ShapeDtypeStruct((N, H), jnp.float32),
            jax.ShapeDtypeStruct((N, 1), jnp.float32),
        ],
    )(x, wc, deg2)


def _finish_body(acc_ref, g_ref, dinv_ref, bc_ref, w1_ref, b1_ref,
                 w2_ref, b2_ref, w3_ref, b3_ref, out_ref, sum_s):
    i = pl.program_id(0)
    a = acc_ref[0] + acc_ref[1] + g_ref[...]
    node = a * dinv_ref[...] + bc_ref[...]
    node = jnp.where(node > 0, node, 0.01 * node)
    h1 = jnp.maximum(
        jnp.dot(node, w1_ref[...], preferred_element_type=jnp.float32)
        + b1_ref[...], 0.0)
    h2 = jnp.maximum(
        jnp.dot(h1, w2_ref[...], preferred_element_type=jnp.float32)
        + b2_ref[...], 0.0)
    part = jnp.sum(h2, axis=0, keepdims=True)

    @pl.when(i == 0)
    def _():
        sum_s[...] = jnp.zeros_like(sum_s)

    sum_s[...] += part

    @pl.when(i == pl.num_programs(0) - 1)
    def _():
        m = sum_s[...] * (1.0 / N)
        z = jnp.dot(m, w3_ref[...], preferred_element_type=jnp.float32) \
            + b3_ref[...]
        out_ref[...] = jax.nn.sigmoid(z)


def _tc_finish(acc, g, dinv, bc, w1, b1, w2, b2, w3, b3):
    blk = 1000
    grid = (N // blk,)
    return pl.pallas_call(
        _finish_body,
        grid=grid,
        in_specs=[
            pl.BlockSpec((NC, blk, H), lambda i: (0, i, 0)),
            pl.BlockSpec((blk, H), lambda i: (i, 0)),
            pl.BlockSpec((blk, 1), lambda i: (i, 0)),
            pl.BlockSpec((1, H), lambda i: (0, 0)),
            pl.BlockSpec((H, H), lambda i: (0, 0)),
            pl.BlockSpec((1, H), lambda i: (0, 0)),
            pl.BlockSpec((H, H), lambda i: (0, 0)),
            pl.BlockSpec((1, H), lambda i: (0, 0)),
            pl.BlockSpec((H, 1), lambda i: (0, 0)),
            pl.BlockSpec((1, 1), lambda i: (0, 0)),
        ],
        out_specs=pl.BlockSpec((1, 1), lambda i: (0, 0)),
        out_shape=jax.ShapeDtypeStruct((1, 1), jnp.float32),
        scratch_shapes=[pltpu.VMEM((1, H), jnp.float32)],
    )(acc, g, dinv, bc, w1, b1, w2, b2, w3, b3)


def kernel(x, edge_index, Wc, bc, W1, b1, W2, b2, W3, b3):
    src = edge_index[0].astype(jnp.int32)
    dst = edge_index[1].astype(jnp.int32)
    pad = E_PAD - E
    src_p = jnp.concatenate(
        [src, jnp.zeros((pad,), jnp.int32)]).reshape(TOT_CHUNKS, CHUNK)
    pad_dst = N + (jnp.arange(pad, dtype=jnp.int32) % (N_ACC - N))
    dst_p = jnp.concatenate([dst, pad_dst]).reshape(TOT_CHUNKS, CHUNK)

    zeros1 = jnp.zeros((N_ACC,), jnp.float32)
    zeros2 = jnp.zeros((N_ACC, H), jnp.float32)

    deg_part = _sc_degree(dst_p, zeros1)
    deg2 = deg_part[:, :N].T
    g, dinv = _tc_prescale(x, Wc, deg2)
    acc = _sc_edges(src_p, dst_p, g, zeros2)

    out = _tc_finish(acc, g, dinv,
                     bc.reshape(1, H), W1, b1.reshape(1, H),
                     W2, b2.reshape(1, H), W3, b3.reshape(1, 1))
    return out.reshape(1)


# speedup vs baseline: 1.1574x; 1.1574x over previous
import functools

import jax
import jax.numpy as jnp
from jax import lax
from jax.experimental import pallas as pl
from jax.experimental.pallas import tpu as pltpu
from jax.experimental.pallas import tpu_sc as plsc

N = 10000
E = 320000
F_IN = 128
H = 64

NC = 2
NS = 16
NW = NC * NS
CHUNK = 128
CPT = 80
EPT = CPT * CHUNK
E_PAD = NW * EPT
N_ACC = 10240
RPT = N_ACC // NS
DUMMY = N

_mesh = plsc.VectorSubcoreMesh(
    core_axis_name="c", subcore_axis_name="s", num_cores=NC, num_subcores=NS)
_sc_params = pltpu.CompilerParams(use_tc_tiling_on_sc=False)


@functools.partial(
    pl.kernel,
    out_type=jax.ShapeDtypeStruct((NC, N_ACC), jnp.float32),
    mesh=_mesh,
    compiler_params=_sc_params,
    scratch_types=[
        pltpu.VMEM((CPT, CHUNK), jnp.int32),
        pltpu.VMEM((CHUNK,), jnp.float32),
        pltpu.VMEM_SHARED((N_ACC,), jnp.float32),
    ],
)
def _sc_degree(dst_hbm, zeros_hbm, out_hbm, dst_v, ones_v, deg_tab):
    c = lax.axis_index("c")
    s = lax.axis_index("s")
    wid = c * NS + s
    pltpu.sync_copy(zeros_hbm.at[pl.ds(s * RPT, RPT)],
                    deg_tab.at[pl.ds(s * RPT, RPT)])
    pltpu.sync_copy(dst_hbm.at[pl.ds(wid * CPT, CPT)], dst_v)
    for i in range(CHUNK // 16):
        ones_v[pl.ds(i * 16, 16)] = jnp.ones((16,), jnp.float32)
    plsc.subcore_barrier()

    def body(j, carry):
        pltpu.sync_copy(ones_v, deg_tab.at[dst_v.at[j]], add=True)
        return carry

    lax.fori_loop(0, CPT, body, 0)
    plsc.subcore_barrier()
    pltpu.sync_copy(deg_tab.at[pl.ds(s * RPT, RPT)],
                    out_hbm.at[c, pl.ds(s * RPT, RPT)])


NBUF = 8
TOT_CHUNKS = E_PAD // CHUNK
K0 = 120
K1 = (TOT_CHUNKS - NS * K0) // NS
KMAX = max(K0, K1)


@functools.partial(
    pl.kernel,
    out_type=jax.ShapeDtypeStruct((NC, N_ACC, H), jnp.float32),
    mesh=_mesh,
    compiler_params=_sc_params,
    scratch_types=[
        pltpu.VMEM((NBUF, CHUNK), jnp.int32),
        pltpu.VMEM((NBUF, CHUNK), jnp.int32),
        pltpu.VMEM((NBUF, CHUNK, H), jnp.float32),
        pltpu.VMEM_SHARED((N_ACC, H), jnp.float32),
    ] + [pltpu.SemaphoreType.DMA] * NBUF,
)
def _sc_edges(src_hbm, dst_hbm, g_hbm, zeros_hbm, out_hbm,
              src_v, dst_v, bufs, acc_tab, *sems):
    c = lax.axis_index("c")
    s = lax.axis_index("s")
    base = jnp.where(c == 0, s * K0, NS * K0 + s * K1)
    cnt = jnp.where(c == 0, K0, K1)
    pltpu.sync_copy(zeros_hbm.at[pl.ds(s * RPT, RPT)],
                    acc_tab.at[pl.ds(s * RPT, RPT)])
    plsc.subcore_barrier()

    def body(t, carry):
        gb = base + t * NBUF
        pltpu.sync_copy(src_hbm.at[pl.ds(gb, NBUF)], src_v)
        pltpu.sync_copy(dst_hbm.at[pl.ds(gb, NBUF)], dst_v)
        descs = [pltpu.async_copy(g_hbm.at[src_v.at[b]],
                                  bufs.at[b], sems[b])
                 for b in range(NBUF)]
        for b in range(NBUF):
            descs[b].wait()
            pltpu.sync_copy(bufs.at[b],
                            acc_tab.at[dst_v.at[b]], add=True)
        return carry

    lax.fori_loop(0, cnt // NBUF, body, 0)
    plsc.subcore_barrier()
    pltpu.sync_copy(acc_tab.at[pl.ds(s * RPT, RPT)],
                    out_hbm.at[c, pl.ds(s * RPT, RPT)])


def _prescale_body(x_ref, wc_ref, deg_ref, g_ref, dinv_ref):
    deg = deg_ref[:, 0] + deg_ref[:, 1] + 1.0
    dinv = lax.rsqrt(deg)
    h = jnp.dot(x_ref[...], wc_ref[...], preferred_element_type=jnp.float32)
    g_ref[...] = h * dinv[:, None]
    dinv_ref[...] = dinv[:, None]


def _tc_prescale(x, wc, deg2):
    blk = 1000
    grid = (N // blk,)
    return pl.pallas_call(
        _prescale_body,
        grid=grid,
        in_specs=[
            pl.BlockSpec((blk, F_IN), lambda i: (i, 0)),
            pl.BlockSpec((F_IN, H), lambda i: (0, 0)),
            pl.BlockSpec((blk, 2), lambda i: (i, 0)),
        ],
        out_specs=[
            pl.BlockSpec((blk, H), lambda i: (i, 0)),
            pl.BlockSpec((blk, 1), lambda i: (i, 0)),
        ],
        out_shape=[
            jax.ShapeDtypeStruct((N, H), jnp.float32),
            jax.ShapeDtypeStruct((N, 1), jnp.float32),
        ],
    )(x, wc, deg2)


def _finish_body(acc_ref, g_ref, dinv_ref, bc_ref, w1_ref, b1_ref,
                 w2_ref, b2_ref, w3_ref, b3_ref, out_ref, sum_s):
    i = pl.program_id(0)
    a = acc_ref[0] + acc_ref[1] + g_ref[...]
    node = a * dinv_ref[...] + bc_ref[...]
    node = jnp.where(node > 0, node, 0.01 * node)
    h1 = jnp.maximum(
        jnp.dot(node, w1_ref[...], preferred_element_type=jnp.float32)
        + b1_ref[...], 0.0)
    h2 = jnp.maximum(
        jnp.dot(h1, w2_ref[...], preferred_element_type=jnp.float32)
        + b2_ref[...], 0.0)
    part = jnp.sum(h2, axis=0, keepdims=True)

    @pl.when(i == 0)
    def _():
        sum_s[...] = jnp.zeros_like(sum_s)

    sum_s[...] += part

    @pl.when(i == pl.num_programs(0) - 1)
    def _():
        m = sum_s[...] * (1.0 / N)
        z = jnp.dot(m, w3_ref[...], preferred_element_type=jnp.float32) \
            + b3_ref[...]
        out_ref[...] = jax.nn.sigmoid(z)


def _tc_finish(acc, g, dinv, bc, w1, b1, w2, b2, w3, b3):
    blk = 1000
    grid = (N // blk,)
    return pl.pallas_call(
        _finish_body,
        grid=grid,
        in_specs=[
            pl.BlockSpec((NC, blk, H), lambda i: (0, i, 0)),
            pl.BlockSpec((blk, H), lambda i: (i, 0)),
            pl.BlockSpec((blk, 1), lambda i: (i, 0)),
            pl.BlockSpec((1, H), lambda i: (0, 0)),
            pl.BlockSpec((H, H), lambda i: (0, 0)),
            pl.BlockSpec((1, H), lambda i: (0, 0)),
            pl.BlockSpec((H, H), lambda i: (0, 0)),
            pl.BlockSpec((1, H), lambda i: (0, 0)),
            pl.BlockSpec((H, 1), lambda i: (0, 0)),
            pl.BlockSpec((1, 1), lambda i: (0, 0)),
        ],
        out_specs=pl.BlockSpec((1, 1), lambda i: (0, 0)),
        out_shape=jax.ShapeDtypeStruct((1, 1), jnp.float32),
        scratch_shapes=[pltpu.VMEM((1, H), jnp.float32)],
    )(acc, g, dinv, bc, w1, b1, w2, b2, w3, b3)


def kernel(x, edge_index, Wc, bc, W1, b1, W2, b2, W3, b3):
    src = edge_index[0].astype(jnp.int32)
    dst = edge_index[1].astype(jnp.int32)
    pad = E_PAD - E
    src_p = jnp.concatenate(
        [src, jnp.zeros((pad,), jnp.int32)]).reshape(TOT_CHUNKS, CHUNK)
    pad_dst = N + (jnp.arange(pad, dtype=jnp.int32) % (N_ACC - N))
    dst_p = jnp.concatenate([dst, pad_dst]).reshape(TOT_CHUNKS, CHUNK)

    zeros1 = jnp.zeros((N_ACC,), jnp.float32)
    zeros2 = jnp.zeros((N_ACC, H), jnp.float32)

    deg_part = _sc_degree(dst_p, zeros1)
    deg2 = deg_part[:, :N].T
    g, dinv = _tc_prescale(x, Wc, deg2)
    acc = _sc_edges(src_p, dst_p, g, zeros2)

    out = _tc_finish(acc, g, dinv,
                     bc.reshape(1, H), W1, b1.reshape(1, H),
                     W2, b2.reshape(1, H), W3, b3.reshape(1, 1))
    return out.reshape(1)


# speedup vs baseline: 1.9342x; 1.6711x over previous
import functools

import jax
import jax.numpy as jnp
from jax import lax
from jax.experimental import pallas as pl
from jax.experimental.pallas import tpu as pltpu
from jax.experimental.pallas import tpu_sc as plsc

N = 10000
E = 320000
F_IN = 128
H = 64

NC = 2
NS = 16
NW = NC * NS
CHUNK = 128
CPT = 80
EPT = CPT * CHUNK
E_PAD = NW * EPT
N_ACC = 10240
RPT = N_ACC // NS
DUMMY = N

_mesh = plsc.VectorSubcoreMesh(
    core_axis_name="c", subcore_axis_name="s", num_cores=NC, num_subcores=NS)
_sc_params = pltpu.CompilerParams(use_tc_tiling_on_sc=False)


@functools.partial(
    pl.kernel,
    out_type=jax.ShapeDtypeStruct((NC, N_ACC), jnp.float32),
    mesh=_mesh,
    compiler_params=_sc_params,
    scratch_types=[
        pltpu.VMEM((CPT, CHUNK), jnp.int32),
        pltpu.VMEM((CHUNK,), jnp.float32),
        pltpu.VMEM_SHARED((N_ACC,), jnp.float32),
    ],
)
def _sc_degree(dst_hbm, zeros_hbm, out_hbm, dst_v, ones_v, deg_tab):
    c = lax.axis_index("c")
    s = lax.axis_index("s")
    wid = c * NS + s
    pltpu.sync_copy(zeros_hbm.at[pl.ds(s * RPT, RPT)],
                    deg_tab.at[pl.ds(s * RPT, RPT)])
    pltpu.sync_copy(dst_hbm.at[pl.ds(wid * CPT, CPT)], dst_v)
    for i in range(CHUNK // 16):
        ones_v[pl.ds(i * 16, 16)] = jnp.ones((16,), jnp.float32)
    plsc.subcore_barrier()

    def body(j, carry):
        pltpu.sync_copy(ones_v, deg_tab.at[dst_v.at[j]], add=True)
        return carry

    lax.fori_loop(0, CPT, body, 0)
    plsc.subcore_barrier()
    pltpu.sync_copy(deg_tab.at[pl.ds(s * RPT, RPT)],
                    out_hbm.at[c, pl.ds(s * RPT, RPT)])


NBUF = 8
TOT_CHUNKS = E_PAD // CHUNK
K0 = 80
K1 = (TOT_CHUNKS - NS * K0) // NS
KMAX = max(K0, K1)


@functools.partial(
    pl.kernel,
    out_type=jax.ShapeDtypeStruct((NC, N_ACC, H), jnp.float32),
    mesh=_mesh,
    compiler_params=_sc_params,
    scratch_types=[
        pltpu.VMEM((NBUF, CHUNK), jnp.int32),
        pltpu.VMEM((NBUF, CHUNK), jnp.int32),
        pltpu.VMEM((NBUF, CHUNK, H), jnp.float32),
        pltpu.VMEM_SHARED((N_ACC, H), jnp.float32),
    ] + [pltpu.SemaphoreType.DMA] * NBUF,
)
def _sc_edges(src_hbm, dst_hbm, g_hbm, zeros_hbm, out_hbm,
              src_v, dst_v, bufs, acc_tab, *sems):
    c = lax.axis_index("c")
    s = lax.axis_index("s")
    base = jnp.where(c == 0, s * K0, NS * K0 + s * K1)
    cnt = jnp.where(c == 0, K0, K1)
    pltpu.sync_copy(zeros_hbm.at[pl.ds(s * RPT, RPT)],
                    acc_tab.at[pl.ds(s * RPT, RPT)])
    plsc.subcore_barrier()

    def body(t, carry):
        gb = base + t * NBUF
        pltpu.sync_copy(src_hbm.at[pl.ds(gb, NBUF)], src_v)
        pltpu.sync_copy(dst_hbm.at[pl.ds(gb, NBUF)], dst_v)
        descs = [pltpu.async_copy(g_hbm.at[src_v.at[b]],
                                  bufs.at[b], sems[b])
                 for b in range(NBUF)]
        for b in range(NBUF):
            descs[b].wait()
            pltpu.sync_copy(bufs.at[b],
                            acc_tab.at[dst_v.at[b]], add=True)
        return carry

    lax.fori_loop(0, cnt // NBUF, body, 0)
    plsc.subcore_barrier()
    pltpu.sync_copy(acc_tab.at[pl.ds(s * RPT, RPT)],
                    out_hbm.at[c, pl.ds(s * RPT, RPT)])


def _prescale_body(x_ref, wc_ref, deg_ref, g_ref, dinv_ref):
    deg = deg_ref[:, 0] + deg_ref[:, 1] + 1.0
    dinv = lax.rsqrt(deg)
    h = jnp.dot(x_ref[...], wc_ref[...], preferred_element_type=jnp.float32)
    g_ref[...] = h * dinv[:, None]
    dinv_ref[...] = dinv[:, None]


def _tc_prescale(x, wc, deg2):
    blk = 1000
    grid = (N // blk,)
    return pl.pallas_call(
        _prescale_body,
        grid=grid,
        in_specs=[
            pl.BlockSpec((blk, F_IN), lambda i: (i, 0)),
            pl.BlockSpec((F_IN, H), lambda i: (0, 0)),
            pl.BlockSpec((blk, 2), lambda i: (i, 0)),
        ],
        out_specs=[
            pl.BlockSpec((blk, H), lambda i: (i, 0)),
            pl.BlockSpec((blk, 1), lambda i: (i, 0)),
        ],
        out_shape=[
            jax.ShapeDtypeStruct((N, H), jnp.float32),
            jax.ShapeDtypeStruct((N, 1), jnp.float32),
        ],
    )(x, wc, deg2)


def _finish_body(acc_ref, g_ref, dinv_ref, bc_ref, w1_ref, b1_ref,
                 w2_ref, b2_ref, w3_ref, b3_ref, out_ref, sum_s):
    i = pl.program_id(0)
    a = acc_ref[0] + acc_ref[1] + g_ref[...]
    node = a * dinv_ref[...] + bc_ref[...]
    node = jnp.where(node > 0, node, 0.01 * node)
    h1 = jnp.maximum(
        jnp.dot(node, w1_ref[...], preferred_element_type=jnp.float32)
        + b1_ref[...], 0.0)
    h2 = jnp.maximum(
        jnp.dot(h1, w2_ref[...], preferred_element_type=jnp.float32)
        + b2_ref[...], 0.0)
    part = jnp.sum(h2, axis=0, keepdims=True)

    @pl.when(i == 0)
    def _():
        sum_s[...] = jnp.zeros_like(sum_s)

    sum_s[...] += part

    @pl.when(i == pl.num_programs(0) - 1)
    def _():
        m = sum_s[...] * (1.0 / N)
        z = jnp.dot(m, w3_ref[...], preferred_element_type=jnp.float32) \
            + b3_ref[...]
        out_ref[...] = jax.nn.sigmoid(z)


def _tc_finish(acc, g, dinv, bc, w1, b1, w2, b2, w3, b3):
    blk = 1000
    grid = (N // blk,)
    return pl.pallas_call(
        _finish_body,
        grid=grid,
        in_specs=[
            pl.BlockSpec((NC, blk, H), lambda i: (0, i, 0)),
            pl.BlockSpec((blk, H), lambda i: (i, 0)),
            pl.BlockSpec((blk, 1), lambda i: (i, 0)),
            pl.BlockSpec((1, H), lambda i: (0, 0)),
            pl.BlockSpec((H, H), lambda i: (0, 0)),
            pl.BlockSpec((1, H), lambda i: (0, 0)),
            pl.BlockSpec((H, H), lambda i: (0, 0)),
            pl.BlockSpec((1, H), lambda i: (0, 0)),
            pl.BlockSpec((H, 1), lambda i: (0, 0)),
            pl.BlockSpec((1, 1), lambda i: (0, 0)),
        ],
        out_specs=pl.BlockSpec((1, 1), lambda i: (0, 0)),
        out_shape=jax.ShapeDtypeStruct((1, 1), jnp.float32),
        scratch_shapes=[pltpu.VMEM((1, H), jnp.float32)],
    )(acc, g, dinv, bc, w1, b1, w2, b2, w3, b3)


def kernel(x, edge_index, Wc, bc, W1, b1, W2, b2, W3, b3):
    src = edge_index[0].astype(jnp.int32)
    dst = edge_index[1].astype(jnp.int32)
    pad = E_PAD - E
    pad_iota = jnp.arange(pad, dtype=jnp.int32)
    src_p = jnp.concatenate(
        [src, pad_iota % N]).reshape(TOT_CHUNKS, CHUNK)
    pad_dst = N + (pad_iota % (N_ACC - N))
    dst_p = jnp.concatenate([dst, pad_dst]).reshape(TOT_CHUNKS, CHUNK)

    zeros1 = jnp.zeros((N_ACC,), jnp.float32)
    zeros2 = jnp.zeros((N_ACC, H), jnp.float32)

    deg_part = _sc_degree(dst_p, zeros1)
    deg2 = deg_part[:, :N].T
    g, dinv = _tc_prescale(x, Wc, deg2)
    acc = _sc_edges(src_p, dst_p, g, zeros2)

    out = _tc_finish(acc, g, dinv,
                     bc.reshape(1, H), W1, b1.reshape(1, H),
                     W2, b2.reshape(1, H), W3, b3.reshape(1, 1))
    return out.reshape(1)


# speedup vs baseline: 2.0327x; 1.0509x over previous
import functools

import jax
import jax.numpy as jnp
from jax import lax
from jax.experimental import pallas as pl
from jax.experimental.pallas import tpu as pltpu
from jax.experimental.pallas import tpu_sc as plsc

N = 10000
E = 320000
F_IN = 128
H = 64

NC = 2
NS = 16
NW = NC * NS
CHUNK = 125
CPT = 80
TOT_CHUNKS = E // CHUNK
N_ACC = 10240
RPT = N_ACC // NS

_mesh = plsc.VectorSubcoreMesh(
    core_axis_name="c", subcore_axis_name="s", num_cores=NC, num_subcores=NS)
_sc_params = pltpu.CompilerParams(use_tc_tiling_on_sc=False)


@functools.partial(
    pl.kernel,
    out_type=jax.ShapeDtypeStruct((NC, N_ACC), jnp.float32),
    mesh=_mesh,
    compiler_params=_sc_params,
    scratch_types=[
        pltpu.VMEM((CPT, CHUNK), jnp.int32),
        pltpu.VMEM((128,), jnp.float32),
        pltpu.VMEM_SHARED((N_ACC,), jnp.float32),
    ],
)
def _sc_degree(dst_hbm, zeros_hbm, out_hbm, dst_v, ones_v, deg_tab):
    c = lax.axis_index("c")
    s = lax.axis_index("s")
    wid = c * NS + s
    pltpu.sync_copy(zeros_hbm.at[pl.ds(s * RPT, RPT)],
                    deg_tab.at[pl.ds(s * RPT, RPT)])
    pltpu.sync_copy(dst_hbm.at[pl.ds(wid * CPT, CPT)], dst_v)
    for i in range(8):
        ones_v[pl.ds(i * 16, 16)] = jnp.ones((16,), jnp.float32)
    plsc.subcore_barrier()

    def body(j, carry):
        pltpu.sync_copy(ones_v.at[pl.ds(0, CHUNK)],
                        deg_tab.at[dst_v.at[j]], add=True)
        return carry

    lax.fori_loop(0, CPT, body, 0)
    plsc.subcore_barrier()
    pltpu.sync_copy(deg_tab.at[pl.ds(s * RPT, RPT)],
                    out_hbm.at[c, pl.ds(s * RPT, RPT)])


NBUF = 10


@functools.partial(
    pl.kernel,
    out_type=jax.ShapeDtypeStruct((NC, N_ACC, H), jnp.float32),
    mesh=_mesh,
    compiler_params=_sc_params,
    scratch_types=[
        pltpu.VMEM((NBUF, CHUNK), jnp.int32),
        pltpu.VMEM((NBUF, CHUNK), jnp.int32),
        pltpu.VMEM((NBUF, CHUNK, H), jnp.float32),
        pltpu.VMEM_SHARED((N_ACC, H), jnp.float32),
    ] + [pltpu.SemaphoreType.DMA] * NBUF,
)
def _sc_edges(src_hbm, dst_hbm, g_hbm, zeros_hbm, out_hbm,
              src_v, dst_v, bufs, acc_tab, *sems):
    c = lax.axis_index("c")
    s = lax.axis_index("s")
    base = (c * NS + s) * CPT
    pltpu.sync_copy(zeros_hbm.at[pl.ds(s * RPT, RPT)],
                    acc_tab.at[pl.ds(s * RPT, RPT)])
    plsc.subcore_barrier()

    def body(t, carry):
        gb = base + t * NBUF
        pltpu.sync_copy(src_hbm.at[pl.ds(gb, NBUF)], src_v)
        pltpu.sync_copy(dst_hbm.at[pl.ds(gb, NBUF)], dst_v)
        descs = [pltpu.async_copy(g_hbm.at[src_v.at[b]],
                                  bufs.at[b], sems[b])
                 for b in range(NBUF)]
        for b in range(NBUF):
            descs[b].wait()
            pltpu.sync_copy(bufs.at[b],
                            acc_tab.at[dst_v.at[b]], add=True)
        return carry

    lax.fori_loop(0, CPT // NBUF, body, 0)
    plsc.subcore_barrier()
    pltpu.sync_copy(acc_tab.at[pl.ds(s * RPT, RPT)],
                    out_hbm.at[c, pl.ds(s * RPT, RPT)])


def _prescale_body(x_ref, wc_ref, deg_ref, g_ref, dinv_ref):
    deg = deg_ref[:, 0] + deg_ref[:, 1] + 1.0
    dinv = lax.rsqrt(deg)
    h = jnp.dot(x_ref[...], wc_ref[...], preferred_element_type=jnp.float32)
    g_ref[...] = h * dinv[:, None]
    dinv_ref[...] = dinv[:, None]


def _tc_prescale(x, wc, deg2):
    blk = 1000
    grid = (N // blk,)
    return pl.pallas_call(
        _prescale_body,
        grid=grid,
        in_specs=[
            pl.BlockSpec((blk, F_IN), lambda i: (i, 0)),
            pl.BlockSpec((F_IN, H), lambda i: (0, 0)),
            pl.BlockSpec((blk, 2), lambda i: (i, 0)),
        ],
        out_specs=[
            pl.BlockSpec((blk, H), lambda i: (i, 0)),
            pl.BlockSpec((blk, 1), lambda i: (i, 0)),
        ],
        out_shape=[
            jax.ShapeDtypeStruct((N, H), jnp.float32),
            jax.ShapeDtypeStruct((N, 1), jnp.float32),
        ],
    )(x, wc, deg2)


def _finish_body(acc_ref, g_ref, dinv_ref, bc_ref, w1_ref, b1_ref,
                 w2_ref, b2_ref, w3_ref, b3_ref, out_ref, sum_s):
    i = pl.program_id(0)
    a = acc_ref[0] + acc_ref[1] + g_ref[...]
    node = a * dinv_ref[...] + bc_ref[...]
    node = jnp.where(node > 0, node, 0.01 * node)
    h1 = jnp.maximum(
        jnp.dot(node, w1_ref[...], preferred_element_type=jnp.float32)
        + b1_ref[...], 0.0)
    h2 = jnp.maximum(
        jnp.dot(h1, w2_ref[...], preferred_element_type=jnp.float32)
        + b2_ref[...], 0.0)
    part = jnp.sum(h2, axis=0, keepdims=True)

    @pl.when(i == 0)
    def _():
        sum_s[...] = jnp.zeros_like(sum_s)

    sum_s[...] += part

    @pl.when(i == pl.num_programs(0) - 1)
    def _():
        m = sum_s[...] * (1.0 / N)
        z = jnp.dot(m, w3_ref[...], preferred_element_type=jnp.float32) \
            + b3_ref[...]
        out_ref[...] = jax.nn.sigmoid(z)


def _tc_finish(acc, g, dinv, bc, w1, b1, w2, b2, w3, b3):
    blk = 1000
    grid = (N // blk,)
    return pl.pallas_call(
        _finish_body,
        grid=grid,
        in_specs=[
            pl.BlockSpec((NC, blk, H), lambda i: (0, i, 0)),
            pl.BlockSpec((blk, H), lambda i: (i, 0)),
            pl.BlockSpec((blk, 1), lambda i: (i, 0)),
            pl.BlockSpec((1, H), lambda i: (0, 0)),
            pl.BlockSpec((H, H), lambda i: (0, 0)),
            pl.BlockSpec((1, H), lambda i: (0, 0)),
            pl.BlockSpec((H, H), lambda i: (0, 0)),
            pl.BlockSpec((1, H), lambda i: (0, 0)),
            pl.BlockSpec((H, 1), lambda i: (0, 0)),
            pl.BlockSpec((1, 1), lambda i: (0, 0)),
        ],
        out_specs=pl.BlockSpec((1, 1), lambda i: (0, 0)),
        out_shape=jax.ShapeDtypeStruct((1, 1), jnp.float32),
        scratch_shapes=[pltpu.VMEM((1, H), jnp.float32)],
    )(acc, g, dinv, bc, w1, b1, w2, b2, w3, b3)


def kernel(x, edge_index, Wc, bc, W1, b1, W2, b2, W3, b3):
    src_p = edge_index[0].astype(jnp.int32).reshape(TOT_CHUNKS, CHUNK)
    dst_p = edge_index[1].astype(jnp.int32).reshape(TOT_CHUNKS, CHUNK)

    zeros1 = jnp.zeros((N_ACC,), jnp.float32)
    zeros2 = jnp.zeros((N_ACC, H), jnp.float32)

    deg_part = _sc_degree(dst_p, zeros1)
    deg2 = deg_part[:, :N].T
    g, dinv = _tc_prescale(x, Wc, deg2)
    acc = _sc_edges(src_p, dst_p, g, zeros2)

    out = _tc_finish(acc, g, dinv,
                     bc.reshape(1, H), W1, b1.reshape(1, H),
                     W2, b2.reshape(1, H), W3, b3.reshape(1, 1))
    return out.reshape(1)


# speedup vs baseline: 2.1180x; 1.0420x over previous
import functools

import jax
import jax.numpy as jnp
from jax import lax
from jax.experimental import pallas as pl
from jax.experimental.pallas import tpu as pltpu
from jax.experimental.pallas import tpu_sc as plsc

N = 10000
E = 320000
F_IN = 128
H = 64

NC = 2
NS = 16
NW = NC * NS
CHUNK = 125
CPT = 80
TOT_CHUNKS = E // CHUNK
N_ACC = 10240
RPT = N_ACC // NS

_mesh = plsc.VectorSubcoreMesh(
    core_axis_name="c", subcore_axis_name="s", num_cores=NC, num_subcores=NS)
_sc_params = pltpu.CompilerParams(use_tc_tiling_on_sc=False)


@functools.partial(
    pl.kernel,
    out_type=jax.ShapeDtypeStruct((NC, N_ACC), jnp.float32),
    mesh=_mesh,
    compiler_params=_sc_params,
    scratch_types=[
        pltpu.VMEM((CPT, CHUNK), jnp.int32),
        pltpu.VMEM((128,), jnp.float32),
        pltpu.VMEM_SHARED((N_ACC,), jnp.float32),
    ],
)
def _sc_degree(edge_hbm, zeros_hbm, out_hbm, dst_v, ones_v, deg_tab):
    c = lax.axis_index("c")
    s = lax.axis_index("s")
    wid = c * NS + s
    pltpu.sync_copy(zeros_hbm.at[pl.ds(s * RPT, RPT)],
                    deg_tab.at[pl.ds(s * RPT, RPT)])
    pltpu.sync_copy(edge_hbm.at[1, pl.ds(wid * CPT, CPT)], dst_v)
    for i in range(8):
        ones_v[pl.ds(i * 16, 16)] = jnp.ones((16,), jnp.float32)
    plsc.subcore_barrier()

    def body(j, carry):
        pltpu.sync_copy(ones_v.at[pl.ds(0, CHUNK)],
                        deg_tab.at[dst_v.at[j]], add=True)
        return carry

    lax.fori_loop(0, CPT, body, 0)
    plsc.subcore_barrier()
    pltpu.sync_copy(deg_tab.at[pl.ds(s * RPT, RPT)],
                    out_hbm.at[c, pl.ds(s * RPT, RPT)])


NBUF = 10


@functools.partial(
    pl.kernel,
    out_type=jax.ShapeDtypeStruct((NC, N_ACC, H), jnp.float32),
    mesh=_mesh,
    compiler_params=_sc_params,
    scratch_types=[
        pltpu.VMEM((NBUF, CHUNK), jnp.int32),
        pltpu.VMEM((NBUF, CHUNK), jnp.int32),
        pltpu.VMEM((NBUF, CHUNK, H), jnp.float32),
        pltpu.VMEM_SHARED((N_ACC, H), jnp.float32),
    ] + [pltpu.SemaphoreType.DMA] * NBUF,
)
def _sc_edges(edge_hbm, g_hbm, zeros_hbm, out_hbm,
              src_v, dst_v, bufs, acc_tab, *sems):
    c = lax.axis_index("c")
    s = lax.axis_index("s")
    base = (c * NS + s) * CPT
    pltpu.sync_copy(zeros_hbm.at[pl.ds(s * RPT, RPT)],
                    acc_tab.at[pl.ds(s * RPT, RPT)])
    plsc.subcore_barrier()

    def body(t, carry):
        gb = base + t * NBUF
        pltpu.sync_copy(edge_hbm.at[0, pl.ds(gb, NBUF)], src_v)
        pltpu.sync_copy(edge_hbm.at[1, pl.ds(gb, NBUF)], dst_v)
        descs = [pltpu.async_copy(g_hbm.at[src_v.at[b]],
                                  bufs.at[b], sems[b])
                 for b in range(NBUF)]
        for b in range(NBUF):
            descs[b].wait()
            pltpu.sync_copy(bufs.at[b],
                            acc_tab.at[dst_v.at[b]], add=True)
        return carry

    lax.fori_loop(0, CPT // NBUF, body, 0)
    plsc.subcore_barrier()
    pltpu.sync_copy(acc_tab.at[pl.ds(s * RPT, RPT)],
                    out_hbm.at[c, pl.ds(s * RPT, RPT)])


def _prescale_body(x_ref, wc_ref, deg_ref, g_ref, dinv_ref):
    deg = deg_ref[:, 0] + deg_ref[:, 1] + 1.0
    dinv = lax.rsqrt(deg)
    h = jnp.dot(x_ref[...], wc_ref[...], preferred_element_type=jnp.float32)
    g_ref[...] = h * dinv[:, None]
    dinv_ref[...] = dinv[:, None]


def _tc_prescale(x, wc, deg2):
    blk = 1000
    grid = (N // blk,)
    return pl.pallas_call(
        _prescale_body,
        grid=grid,
        in_specs=[
            pl.BlockSpec((blk, F_IN), lambda i: (i, 0)),
            pl.BlockSpec((F_IN, H), lambda i: (0, 0)),
            pl.BlockSpec((blk, 2), lambda i: (i, 0)),
        ],
        out_specs=[
            pl.BlockSpec((blk, H), lambda i: (i, 0)),
            pl.BlockSpec((blk, 1), lambda i: (i, 0)),
        ],
        out_shape=[
            jax.ShapeDtypeStruct((N, H), jnp.float32),
            jax.ShapeDtypeStruct((N, 1), jnp.float32),
        ],
    )(x, wc, deg2)


def _finish_body(acc_ref, g_ref, dinv_ref, bc_ref, w1_ref, b1_ref,
                 w2_ref, b2_ref, w3_ref, b3_ref, out_ref, sum_s):
    i = pl.program_id(0)
    a = acc_ref[0] + acc_ref[1] + g_ref[...]
    node = a * dinv_ref[...] + bc_ref[...]
    node = jnp.where(node > 0, node, 0.01 * node)
    h1 = jnp.maximum(
        jnp.dot(node, w1_ref[...], preferred_element_type=jnp.float32)
        + b1_ref[...], 0.0)
    h2 = jnp.maximum(
        jnp.dot(h1, w2_ref[...], preferred_element_type=jnp.float32)
        + b2_ref[...], 0.0)
    part = jnp.sum(h2, axis=0, keepdims=True)

    @pl.when(i == 0)
    def _():
        sum_s[...] = jnp.zeros_like(sum_s)

    sum_s[...] += part

    @pl.when(i == pl.num_programs(0) - 1)
    def _():
        m = sum_s[...] * (1.0 / N)
        z = jnp.dot(m, w3_ref[...], preferred_element_type=jnp.float32) \
            + b3_ref[...]
        out_ref[...] = jax.nn.sigmoid(z)


def _tc_finish(acc, g, dinv, bc, w1, b1, w2, b2, w3, b3):
    blk = 1000
    grid = (N // blk,)
    return pl.pallas_call(
        _finish_body,
        grid=grid,
        in_specs=[
            pl.BlockSpec((NC, blk, H), lambda i: (0, i, 0)),
            pl.BlockSpec((blk, H), lambda i: (i, 0)),
            pl.BlockSpec((blk, 1), lambda i: (i, 0)),
            pl.BlockSpec((1, H), lambda i: (0, 0)),
            pl.BlockSpec((H, H), lambda i: (0, 0)),
            pl.BlockSpec((1, H), lambda i: (0, 0)),
            pl.BlockSpec((H, H), lambda i: (0, 0)),
            pl.BlockSpec((1, H), lambda i: (0, 0)),
            pl.BlockSpec((H, 1), lambda i: (0, 0)),
            pl.BlockSpec((1, 1), lambda i: (0, 0)),
        ],
        out_specs=pl.BlockSpec((1, 1), lambda i: (0, 0)),
        out_shape=jax.ShapeDtypeStruct((1, 1), jnp.float32),
        scratch_shapes=[pltpu.VMEM((1, H), jnp.float32)],
    )(acc, g, dinv, bc, w1, b1, w2, b2, w3, b3)


def kernel(x, edge_index, Wc, bc, W1, b1, W2, b2, W3, b3):
    edge3 = edge_index.astype(jnp.int32).reshape(2, TOT_CHUNKS, CHUNK)

    zeros1 = jnp.zeros((N_ACC,), jnp.float32)
    zeros2 = jnp.zeros((N_ACC, H), jnp.float32)

    deg_part = _sc_degree(edge3, zeros1)
    deg2 = deg_part[:, :N].T
    g, dinv = _tc_prescale(x, Wc, deg2)
    acc = _sc_edges(edge3, g, zeros2)

    out = _tc_finish(acc, g, dinv,
                     bc.reshape(1, H), W1, b1.reshape(1, H),
                     W2, b2.reshape(1, H), W3, b3.reshape(1, 1))
    return out.reshape(1)


# speedup vs baseline: 2.1907x; 1.0343x over previous
import functools

import jax
import jax.numpy as jnp
from jax import lax
from jax.experimental import pallas as pl
from jax.experimental.pallas import tpu as pltpu
from jax.experimental.pallas import tpu_sc as plsc

N = 10000
E = 320000
F_IN = 128
H = 64

NC = 2
NS = 16
NW = NC * NS
CHUNK = 125
CPT = 80
TOT_CHUNKS = E // CHUNK
N_ACC = 10240
RPT = N_ACC // NS

_mesh = plsc.VectorSubcoreMesh(
    core_axis_name="c", subcore_axis_name="s", num_cores=NC, num_subcores=NS)
_sc_params = pltpu.CompilerParams(use_tc_tiling_on_sc=False)


@functools.partial(
    pl.kernel,
    out_type=jax.ShapeDtypeStruct((NC, N_ACC), jnp.float32),
    mesh=_mesh,
    compiler_params=_sc_params,
    scratch_types=[
        pltpu.VMEM((CPT, CHUNK), jnp.int32),
        pltpu.VMEM((128,), jnp.float32),
        pltpu.VMEM_SHARED((N_ACC,), jnp.float32),
    ],
)
def _sc_degree(edge_hbm, zeros_hbm, out_hbm, dst_v, ones_v, deg_tab):
    c = lax.axis_index("c")
    s = lax.axis_index("s")
    wid = c * NS + s
    pltpu.sync_copy(zeros_hbm.at[pl.ds(s * RPT, RPT)],
                    deg_tab.at[pl.ds(s * RPT, RPT)])
    pltpu.sync_copy(edge_hbm.at[1, pl.ds(wid * CPT, CPT)], dst_v)
    for i in range(8):
        ones_v[pl.ds(i * 16, 16)] = jnp.ones((16,), jnp.float32)
    plsc.subcore_barrier()

    def body(j, carry):
        pltpu.sync_copy(ones_v.at[pl.ds(0, CHUNK)],
                        deg_tab.at[dst_v.at[j]], add=True)
        return carry

    lax.fori_loop(0, CPT, body, 0)
    plsc.subcore_barrier()
    pltpu.sync_copy(deg_tab.at[pl.ds(s * RPT, RPT)],
                    out_hbm.at[c, pl.ds(s * RPT, RPT)])


NBUF = 8


@functools.partial(
    pl.kernel,
    out_type=jax.ShapeDtypeStruct((NC, N_ACC, H), jnp.float32),
    mesh=_mesh,
    compiler_params=_sc_params,
    scratch_types=[
        pltpu.VMEM((CPT, CHUNK), jnp.int32),
        pltpu.VMEM((CPT, CHUNK), jnp.int32),
        pltpu.VMEM((NBUF, CHUNK, H), jnp.float32),
        pltpu.VMEM_SHARED((N_ACC, H), jnp.float32),
    ] + [pltpu.SemaphoreType.DMA] * NBUF,
)
def _sc_edges(edge_hbm, g_hbm, zeros_hbm, out_hbm,
              src_v, dst_v, bufs, acc_tab, *sems):
    c = lax.axis_index("c")
    s = lax.axis_index("s")
    base = (c * NS + s) * CPT
    pltpu.sync_copy(edge_hbm.at[0, pl.ds(base, CPT)], src_v)
    pltpu.sync_copy(edge_hbm.at[1, pl.ds(base, CPT)], dst_v)
    pltpu.sync_copy(zeros_hbm.at[pl.ds(s * RPT, RPT)],
                    acc_tab.at[pl.ds(s * RPT, RPT)])
    plsc.subcore_barrier()

    def body(t, carry):
        j = t * NBUF
        descs = [pltpu.async_copy(g_hbm.at[src_v.at[j + b]],
                                  bufs.at[b], sems[b])
                 for b in range(NBUF)]
        for b in range(NBUF):
            descs[b].wait()
            pltpu.sync_copy(bufs.at[b],
                            acc_tab.at[dst_v.at[j + b]], add=True)
        return carry

    lax.fori_loop(0, CPT // NBUF, body, 0)
    plsc.subcore_barrier()
    pltpu.sync_copy(acc_tab.at[pl.ds(s * RPT, RPT)],
                    out_hbm.at[c, pl.ds(s * RPT, RPT)])


def _prescale_body(x_ref, wc_ref, deg_ref, g_ref, dinv_ref):
    deg = deg_ref[:, 0] + deg_ref[:, 1] + 1.0
    dinv = lax.rsqrt(deg)
    h = jnp.dot(x_ref[...], wc_ref[...], preferred_element_type=jnp.float32)
    g_ref[...] = h * dinv[:, None]
    dinv_ref[...] = dinv[:, None]


def _tc_prescale(x, wc, deg2):
    blk = 1000
    grid = (N // blk,)
    return pl.pallas_call(
        _prescale_body,
        grid=grid,
        in_specs=[
            pl.BlockSpec((blk, F_IN), lambda i: (i, 0)),
            pl.BlockSpec((F_IN, H), lambda i: (0, 0)),
            pl.BlockSpec((blk, 2), lambda i: (i, 0)),
        ],
        out_specs=[
            pl.BlockSpec((blk, H), lambda i: (i, 0)),
            pl.BlockSpec((blk, 1), lambda i: (i, 0)),
        ],
        out_shape=[
            jax.ShapeDtypeStruct((N, H), jnp.float32),
            jax.ShapeDtypeStruct((N, 1), jnp.float32),
        ],
    )(x, wc, deg2)


def _finish_body(acc_ref, g_ref, dinv_ref, bc_ref, w1_ref, b1_ref,
                 w2_ref, b2_ref, w3_ref, b3_ref, out_ref, sum_s):
    i = pl.program_id(0)
    a = acc_ref[0] + acc_ref[1] + g_ref[...]
    node = a * dinv_ref[...] + bc_ref[...]
    node = jnp.where(node > 0, node, 0.01 * node)
    h1 = jnp.maximum(
        jnp.dot(node, w1_ref[...], preferred_element_type=jnp.float32)
        + b1_ref[...], 0.0)
    h2 = jnp.maximum(
        jnp.dot(h1, w2_ref[...], preferred_element_type=jnp.float32)
        + b2_ref[...], 0.0)
    part = jnp.sum(h2, axis=0, keepdims=True)

    @pl.when(i == 0)
    def _():
        sum_s[...] = jnp.zeros_like(sum_s)

    sum_s[...] += part

    @pl.when(i == pl.num_programs(0) - 1)
    def _():
        m = sum_s[...] * (1.0 / N)
        z = jnp.dot(m, w3_ref[...], preferred_element_type=jnp.float32) \
            + b3_ref[...]
        out_ref[...] = jax.nn.sigmoid(z)


def _tc_finish(acc, g, dinv, bc, w1, b1, w2, b2, w3, b3):
    blk = 1000
    grid = (N // blk,)
    return pl.pallas_call(
        _finish_body,
        grid=grid,
        in_specs=[
            pl.BlockSpec((NC, blk, H), lambda i: (0, i, 0)),
            pl.BlockSpec((blk, H), lambda i: (i, 0)),
            pl.BlockSpec((blk, 1), lambda i: (i, 0)),
            pl.BlockSpec((1, H), lambda i: (0, 0)),
            pl.BlockSpec((H, H), lambda i: (0, 0)),
            pl.BlockSpec((1, H), lambda i: (0, 0)),
            pl.BlockSpec((H, H), lambda i: (0, 0)),
            pl.BlockSpec((1, H), lambda i: (0, 0)),
            pl.BlockSpec((H, 1), lambda i: (0, 0)),
            pl.BlockSpec((1, 1), lambda i: (0, 0)),
        ],
        out_specs=pl.BlockSpec((1, 1), lambda i: (0, 0)),
        out_shape=jax.ShapeDtypeStruct((1, 1), jnp.float32),
        scratch_shapes=[pltpu.VMEM((1, H), jnp.float32)],
    )(acc, g, dinv, bc, w1, b1, w2, b2, w3, b3)


def kernel(x, edge_index, Wc, bc, W1, b1, W2, b2, W3, b3):
    edge3 = edge_index.astype(jnp.int32).reshape(2, TOT_CHUNKS, CHUNK)

    zeros1 = jnp.zeros((N_ACC,), jnp.float32)
    zeros2 = jnp.zeros((N_ACC, H), jnp.float32)

    deg_part = _sc_degree(edge3, zeros1)
    deg2 = deg_part[:, :N].T
    g, dinv = _tc_prescale(x, Wc, deg2)
    acc = _sc_edges(edge3, g, zeros2)

    out = _tc_finish(acc, g, dinv,
                     bc.reshape(1, H), W1, b1.reshape(1, H),
                     W2, b2.reshape(1, H), W3, b3.reshape(1, 1))
    return out.reshape(1)
